# baseline (speedup 1.0000x reference)
import jax
import jax.numpy as jnp
from jax import lax
from jax.experimental import pallas as pl
from jax.experimental.pallas import tpu as pltpu
from jax.experimental.pallas import tpu_sc as plsc

N_NODES = 10000
N_EDGES = 320000
D = 128

NC = 2
NS = 16
NW = NC * NS
CHUNK = 128
CPW = 80
SEG = 16
N_SEG = CPW // SEG
N_PAD = NW * CPW * CHUNK - N_EDGES

ROWS_PER_SUB = 624
TAIL_ROWS = N_NODES - NS * ROWS_PER_SUB
ZROWS = 104


def _matmul_body(x_ref, w_ref, o_ref):
    o_ref[...] = jnp.dot(x_ref[...], w_ref[...],
                         preferred_element_type=jnp.float32)


def _matmul(x, W):
    blk = 1000
    return pl.pallas_call(
        _matmul_body,
        grid=(N_NODES // blk,),
        in_specs=[
            pl.BlockSpec((blk, D), lambda i: (i, 0)),
            pl.BlockSpec((D, D), lambda i: (0, 0)),
        ],
        out_specs=pl.BlockSpec((blk, D), lambda i: (i, 0)),
        out_shape=jax.ShapeDtypeStruct((N_NODES, D), jnp.float32),
    )(x, W)


def _combine_body(p_ref, o_ref):
    o_ref[...] = jnp.maximum(p_ref[0] + p_ref[1], 0.0)


def _combine(partials):
    blk = 1000
    return pl.pallas_call(
        _combine_body,
        grid=(N_NODES // blk,),
        in_specs=[pl.BlockSpec((2, blk, D), lambda i: (0, i, 0))],
        out_specs=pl.BlockSpec((blk, D), lambda i: (i, 0)),
        out_shape=jax.ShapeDtypeStruct((N_NODES, D), jnp.float32),
    )(partials)


def _sc_body(support_hbm, src_hbm, dst_hbm, ew_hbm, out_hbm,
             acc_spmem, src_v, dst_v, w_v, rows0, rows1, sem0, sem1):
    core = lax.axis_index("c")
    sub = lax.axis_index("s")
    wid = sub * NC + core

    @pl.loop(0, ZROWS)
    def _(r):
        for g in range(D // 16):
            rows0[r, pl.ds(g * 16, 16)] = jnp.zeros((16,), jnp.float32)

    base = sub * ROWS_PER_SUB
    for k in range(ROWS_PER_SUB // ZROWS):
        pltpu.sync_copy(rows0.at[pl.ds(0, ZROWS)],
                        acc_spmem.at[pl.ds(base + k * ZROWS, ZROWS)])

    @pl.when(sub == NS - 1)
    def _():
        pltpu.sync_copy(rows0.at[pl.ds(0, TAIL_ROWS)],
                        acc_spmem.at[pl.ds(NS * ROWS_PER_SUB, TAIL_ROWS)])

    plsc.subcore_barrier()

    def scale(rows, c):
        @pl.loop(0, CHUNK, step=16)
        def _(eg):
            w16 = w_v[c, pl.ds(eg, 16)]
            for j in range(16):
                bw = jnp.full((16,), w16[j], jnp.float32)
                for g in range(D // 16):
                    sl = pl.ds(g * 16, 16)
                    rows[eg + j, sl] = rows[eg + j, sl] * bw

    for h in range(N_SEG):
        pltpu.sync_copy(src_hbm.at[wid, pl.ds(h * SEG, SEG)], src_v)
        pltpu.sync_copy(dst_hbm.at[wid, pl.ds(h * SEG, SEG)], dst_v)
        pltpu.sync_copy(ew_hbm.at[wid, pl.ds(h * SEG, SEG)], w_v)

        pltpu.async_copy(support_hbm.at[src_v.at[0]], rows0, sem0)

        @pl.loop(0, SEG, step=2)
        def _(c):
            pltpu.async_copy(support_hbm.at[src_v.at[c + 1]], rows1, sem1)
            pltpu.make_async_copy(support_hbm.at[src_v.at[c]],
                                  rows0, sem0).wait()
            scale(rows0, c)
            pltpu.sync_copy(rows0, acc_spmem.at[dst_v.at[c]], add=True)

            @pl.when(c + 2 < SEG)
            def _():
                pltpu.async_copy(support_hbm.at[src_v.at[c + 2]], rows0, sem0)

            pltpu.make_async_copy(support_hbm.at[src_v.at[c + 1]],
                                  rows1, sem1).wait()
            scale(rows1, c + 1)
            pltpu.sync_copy(rows1, acc_spmem.at[dst_v.at[c + 1]], add=True)

    plsc.subcore_barrier()

    ob = core * N_NODES + base
    for k in range(ROWS_PER_SUB // ZROWS):
        pltpu.sync_copy(acc_spmem.at[pl.ds(base + k * ZROWS, ZROWS)],
                        out_hbm.at[pl.ds(ob + k * ZROWS, ZROWS)])

    @pl.when(sub == NS - 1)
    def _():
        pltpu.sync_copy(acc_spmem.at[pl.ds(NS * ROWS_PER_SUB, TAIL_ROWS)],
                        out_hbm.at[pl.ds(core * N_NODES + NS * ROWS_PER_SUB,
                                         TAIL_ROWS)])


def _sc_spmm(support, src, dst, ew):
    mesh = plsc.VectorSubcoreMesh(core_axis_name="c", subcore_axis_name="s")
    f = pl.kernel(
        _sc_body,
        out_type=jax.ShapeDtypeStruct((NC * N_NODES, D), jnp.float32),
        mesh=mesh,
        scratch_types=[
            pltpu.VMEM_SHARED((N_NODES, D), jnp.float32),
            pltpu.VMEM((SEG, CHUNK), jnp.int32),
            pltpu.VMEM((SEG, CHUNK), jnp.int32),
            pltpu.VMEM((SEG, CHUNK), jnp.float32),
            pltpu.VMEM((CHUNK, D), jnp.float32),
            pltpu.VMEM((CHUNK, D), jnp.float32),
            pltpu.SemaphoreType.DMA,
            pltpu.SemaphoreType.DMA,
        ],
    )
    return f(support, src, dst, ew)


def kernel(x, edge_index, edge_weight, W):
    support = _matmul(x, W)
    dst = jnp.pad(edge_index[0], (0, N_PAD)).reshape(NW, CPW, CHUNK)
    src = jnp.pad(edge_index[1], (0, N_PAD)).reshape(NW, CPW, CHUNK)
    ew = jnp.pad(edge_weight, (0, N_PAD)).reshape(NW, CPW, CHUNK)
    partials = _sc_spmm(support, src, dst, ew)
    return _combine(partials.reshape(NC, N_NODES, D))
